# Initial kernel scaffold; baseline (speedup 1.0000x reference)
import jax
import jax.numpy as jnp
from jax.experimental import pallas as pl


def kernel(x, token_table, pos_table):
    raise NotImplementedError("write your pallas kernel here")


# speedup vs baseline: 4.1397x; 4.1397x over previous
import functools

import jax
import jax.numpy as jnp
from jax import lax
from jax.experimental import pallas as pl
from jax.experimental.pallas import tpu as pltpu
from jax.experimental.pallas import tpu_sc as plsc

NC = 2
NS = 16
L = 16
NW = NC * NS


def _make_sc_kernel(n_tokens, maxlen, embed_dim, seqs_per_chunk):
    tok_per_w = n_tokens // NW
    chunk = seqs_per_chunk * maxlen
    n_chunks = tok_per_w // chunk
    assert tok_per_w % chunk == 0
    vregs_per_row = embed_dim // L

    mesh = plsc.VectorSubcoreMesh(core_axis_name="c", subcore_axis_name="s")

    @functools.partial(
        pl.kernel,
        out_type=jax.ShapeDtypeStruct((n_tokens, embed_dim), jnp.float32),
        mesh=mesh,
        scratch_types=[
            pltpu.VMEM((chunk,), jnp.int32),
            pltpu.VMEM((chunk, embed_dim), jnp.float32),
            pltpu.VMEM((maxlen, embed_dim), jnp.float32),
            pltpu.SemaphoreType.DMA,
        ],
        compiler_params=pltpu.CompilerParams(use_tc_tiling_on_sc=False),
    )
    def k(x_hbm, tok_hbm, pos_hbm, out_hbm, idx_v, rows_v, pos_v, sem):
        wid = lax.axis_index("s") * NC + lax.axis_index("c")
        wbase = wid * tok_per_w

        pltpu.sync_copy(pos_hbm, pos_v)

        @pl.loop(0, n_chunks)
        def _chunk_loop(ci):
            base = wbase + ci * chunk
            pltpu.sync_copy(x_hbm.at[pl.ds(base, chunk)], idx_v)
            pltpu.async_copy(tok_hbm.at[idx_v], rows_v, sem).wait()

            @pl.loop(0, seqs_per_chunk)
            def _seq_loop(s):
                @pl.loop(0, maxlen)
                def _tok_loop(t):
                    r = s * maxlen + t
                    for v in range(vregs_per_row):
                        sl = pl.ds(v * L, L)
                        rows_v[r, sl] = rows_v[r, sl] + pos_v[t, sl]

            pltpu.sync_copy(rows_v, out_hbm.at[pl.ds(base, chunk)])

    return k


def kernel(x, token_table, pos_table):
    batch, maxlen = x.shape
    vocab, embed_dim = token_table.shape
    n_tokens = batch * maxlen
    x_flat = x.reshape(n_tokens).astype(jnp.int32)
    k = _make_sc_kernel(n_tokens, maxlen, embed_dim, seqs_per_chunk=4)
    out = k(x_flat, token_table, pos_table)
    return out.reshape(batch, maxlen, embed_dim)


# speedup vs baseline: 4.7177x; 1.1396x over previous
import functools

import jax
import jax.numpy as jnp
from jax import lax
from jax.experimental import pallas as pl
from jax.experimental.pallas import tpu as pltpu
from jax.experimental.pallas import tpu_sc as plsc

NC = 2
NS = 16
L = 16
NW = NC * NS
NB = 4


def _make_sc_kernel(n_tokens, maxlen, embed_dim, seqs_per_chunk):
    tok_per_w = n_tokens // NW
    chunk = seqs_per_chunk * maxlen
    n_chunks = tok_per_w // chunk
    assert n_tokens % NW == 0 and tok_per_w % chunk == 0
    assert n_chunks % NB == 0 and n_chunks >= 2 * NB
    vregs_per_row = embed_dim // L

    mesh = plsc.VectorSubcoreMesh(core_axis_name="c", subcore_axis_name="s")

    @functools.partial(
        pl.kernel,
        out_type=jax.ShapeDtypeStruct((n_tokens, embed_dim), jnp.float32),
        mesh=mesh,
        scratch_types=[
            pltpu.VMEM((tok_per_w,), jnp.int32),
            [pltpu.VMEM((chunk, embed_dim), jnp.float32) for _ in range(NB)],
            pltpu.VMEM((maxlen, embed_dim), jnp.float32),
            [pltpu.SemaphoreType.DMA for _ in range(NB)],
            [pltpu.SemaphoreType.DMA for _ in range(NB)],
        ],
        compiler_params=pltpu.CompilerParams(use_tc_tiling_on_sc=False),
    )
    def k(x_hbm, tok_hbm, pos_hbm, out_hbm, idx_all, rows, pos_v, semg, sems):
        wid = lax.axis_index("s") * NC + lax.axis_index("c")
        wbase = wid * tok_per_w

        pltpu.sync_copy(pos_hbm, pos_v)
        pltpu.sync_copy(x_hbm.at[pl.ds(wbase, tok_per_w)], idx_all)

        def idx_slice(cur):
            return idx_all.at[pl.ds(cur * chunk, chunk)]

        def issue_gather(cur, b):
            pltpu.async_copy(tok_hbm.at[idx_slice(cur)], rows[b], semg[b])

        def wait_gather(cur, b):
            pltpu.make_async_copy(
                tok_hbm.at[idx_slice(cur)], rows[b], semg[b]).wait()

        def issue_store(cur, b):
            base = wbase + cur * chunk
            pltpu.async_copy(rows[b], out_hbm.at[pl.ds(base, chunk)], sems[b])

        def wait_store(cur, b):
            base = wbase + cur * chunk
            pltpu.make_async_copy(
                rows[b], out_hbm.at[pl.ds(base, chunk)], sems[b]).wait()

        def add_pos(rb):
            @pl.loop(0, maxlen, unroll=2)
            def _tok_loop(t):
                for v in range(vregs_per_row):
                    sl = pl.ds(v * L, L)
                    pv = pos_v[t, sl]
                    for s in range(seqs_per_chunk):
                        r = s * maxlen + t
                        rb[r, sl] = rb[r, sl] + pv

        def emit(cur, b, wait_prev, issue_next):
            wait_gather(cur, b)
            add_pos(rows[b])
            issue_store(cur, b)
            br = (b + NB - 1) % NB
            if wait_prev:
                wait_store(cur - 1, br)
            if issue_next:
                issue_gather(cur + NB - 1, br)

        for c in range(NB - 1):
            issue_gather(c, c)

        for c in range(NB):
            emit(c, c, wait_prev=(c >= 1), issue_next=True)

        @pl.loop(NB, n_chunks - NB, step=NB)
        def _chunk_loop(ci):
            for j in range(NB):
                emit(ci + j, j, wait_prev=True, issue_next=True)

        for c in range(n_chunks - NB, n_chunks):
            go = c + NB - 1 < n_chunks
            emit(c, c % NB, wait_prev=go, issue_next=go)

        for c in range(n_chunks - NB, n_chunks):
            wait_store(c, c % NB)

    return k


def kernel(x, token_table, pos_table):
    batch, maxlen = x.shape
    vocab, embed_dim = token_table.shape
    n_tokens = batch * maxlen
    x_flat = x.reshape(n_tokens).astype(jnp.int32)
    k = _make_sc_kernel(n_tokens, maxlen, embed_dim, seqs_per_chunk=2)
    out = k(x_flat, token_table, pos_table)
    return out.reshape(batch, maxlen, embed_dim)
